# Initial kernel scaffold; baseline (speedup 1.0000x reference)
import jax
import jax.numpy as jnp
from jax.experimental import pallas as pl


def kernel(x, edge_index, edge_attr, feature1, feature2, batch, lin_x_W, lin_x_b, lin_x_g, lin_x_be, f1_W1, f1_b1, f1_g, f1_be, f1_W2, f1_b2, f2_W1, f2_b1, f2_g, f2_be, f2_W2, f2_b2, c1_rel_W, c1_rel_b, c1_root_W, cat_W, cat_b, l0_W, l0_b, l1_W, l1_b, l2_W, l2_b, gn_w, gn_b, gn_ms, fin_W, fin_b):
    raise NotImplementedError("write your pallas kernel here")


# speedup vs baseline: 1.6504x; 1.6504x over previous
import functools

import jax
import jax.numpy as jnp
from jax import lax
from jax.experimental import pallas as pl
from jax.experimental.pallas import tpu as pltpu
from jax.experimental.pallas import tpu_sc as plsc

N = 10000
E = 160000
H = 256
G = 64
NP = 10240
RB = 1000
EB = 1600
GB = 2000
KE = 80
EPT = E // 16
NCH = EPT // KE
ZR = 640
EPS = 1e-5


def _mm(a, b):
    return jax.lax.dot_general(a, b, (((1,), (0,)), ((), ())),
                               preferred_element_type=jnp.float32)


def _mmT(a, b):
    return jax.lax.dot_general(a, b, (((0,), (0,)), ((), ())),
                               preferred_element_type=jnp.float32)


def _a1_body(x_ref, w_ref, z_ref, st_ref):
    i = pl.program_id(0)
    z = _mm(x_ref[...], w_ref[...])
    z_ref[...] = z

    @pl.when(i == 0)
    def _():
        st_ref[...] = jnp.zeros_like(st_ref)

    st_ref[0:1, :] += jnp.sum(z, axis=0, keepdims=True)
    st_ref[1:2, :] += jnp.sum(z * z, axis=0, keepdims=True)


def _a2_body(z_ref, st_ref, g_ref, be_ref, xt_ref, xts_ref):
    mu = st_ref[0:1, :] / N
    var = st_ref[1:2, :] / N - mu * mu
    xt = jnp.maximum(g_ref[...] * (z_ref[...] - mu) * lax.rsqrt(var + EPS)
                     + be_ref[...], 0.0)
    xt_ref[...] = xt
    xts_ref[0] = xt[:, :128]
    xts_ref[1] = xt[:, 128:]


def _stage_a(x, w, g, be):
    z, st = pl.pallas_call(
        _a1_body,
        grid=(N // RB,),
        in_specs=[pl.BlockSpec((RB, H), lambda i: (i, 0)),
                  pl.BlockSpec((H, H), lambda i: (0, 0))],
        out_specs=[pl.BlockSpec((RB, H), lambda i: (i, 0)),
                   pl.BlockSpec((8, H), lambda i: (0, 0))],
        out_shape=[jax.ShapeDtypeStruct((N, H), jnp.float32),
                   jax.ShapeDtypeStruct((8, H), jnp.float32)],
    )(x, w)
    xt, xts = pl.pallas_call(
        _a2_body,
        grid=(N // RB,),
        in_specs=[pl.BlockSpec((RB, H), lambda i: (i, 0)),
                  pl.BlockSpec((8, H), lambda i: (0, 0)),
                  pl.BlockSpec((1, H), lambda i: (0, 0)),
                  pl.BlockSpec((1, H), lambda i: (0, 0))],
        out_specs=[pl.BlockSpec((RB, H), lambda i: (i, 0)),
                   pl.BlockSpec((2, RB, 128), lambda i: (0, i, 0))],
        out_shape=[jax.ShapeDtypeStruct((N, H), jnp.float32),
                   jax.ShapeDtypeStruct((2, N, 128), jnp.float32)],
    )(z, st, g, be)
    return xt, xts


def _b0_body(f_ref, cs_ref, gr_ref):
    i = pl.program_id(0)
    blk = f_ref[...]

    @pl.when(i == 0)
    def _():
        cs_ref[...] = jnp.zeros_like(cs_ref)
        gr_ref[...] = jnp.zeros_like(gr_ref)

    cs_ref[0:1, :] += jnp.sum(blk, axis=0, keepdims=True)
    gr_ref[...] += _mmT(blk, blk)


def _b1_body(cs_ref, gr_ref, w1_ref, g_ref, be_ref, w1s_ref, b1s_ref):
    mean_f = cs_ref[0:1, :] / E
    mu1 = _mm(mean_f, w1_ref[...])
    cov = gr_ref[...] / E - _mmT(mean_f, mean_f)
    cw = _mm(cov, w1_ref[...])
    var = jnp.sum(cw * w1_ref[...], axis=0, keepdims=True)
    s = g_ref[...] * lax.rsqrt(var + EPS)
    w1s_ref[...] = w1_ref[...] * s
    b1s_ref[...] = be_ref[...] - mu1 * s


def _b2_body(f_ref, w1s_ref, b1s_ref, w2_ref, b2_ref, hfs_ref):
    h1 = jnp.maximum(_mm(f_ref[...], w1s_ref[...]) + b1s_ref[...], 0.0)
    hf = jnp.maximum(_mm(h1, w2_ref[...]) + b2_ref[...], 0.0)
    hfs_ref[0] = hf[:, :128]
    hfs_ref[1] = hf[:, 128:]


def _stage_b(f, w1, g, be, w2, b2):
    dp = f.shape[1]
    cs, gr = pl.pallas_call(
        _b0_body,
        grid=(E // GB,),
        in_specs=[pl.BlockSpec((GB, dp), lambda i: (i, 0))],
        out_specs=[pl.BlockSpec((8, dp), lambda i: (0, 0)),
                   pl.BlockSpec((dp, dp), lambda i: (0, 0))],
        out_shape=[jax.ShapeDtypeStruct((8, dp), jnp.float32),
                   jax.ShapeDtypeStruct((dp, dp), jnp.float32)],
    )(f)
    w1s, b1s = pl.pallas_call(
        _b1_body,
        in_specs=[pl.BlockSpec((8, dp), lambda: (0, 0)),
                  pl.BlockSpec((dp, dp), lambda: (0, 0)),
                  pl.BlockSpec((dp, H), lambda: (0, 0)),
                  pl.BlockSpec((1, H), lambda: (0, 0)),
                  pl.BlockSpec((1, H), lambda: (0, 0))],
        out_specs=[pl.BlockSpec((dp, H), lambda: (0, 0)),
                   pl.BlockSpec((1, H), lambda: (0, 0))],
        out_shape=[jax.ShapeDtypeStruct((dp, H), jnp.float32),
                   jax.ShapeDtypeStruct((1, H), jnp.float32)],
    )(cs, gr, w1, g, be)
    hfs = pl.pallas_call(
        _b2_body,
        grid=(E // EB,),
        in_specs=[pl.BlockSpec((EB, dp), lambda i: (i, 0)),
                  pl.BlockSpec((dp, H), lambda i: (0, 0)),
                  pl.BlockSpec((1, H), lambda i: (0, 0)),
                  pl.BlockSpec((H, H), lambda i: (0, 0)),
                  pl.BlockSpec((1, H), lambda i: (0, 0))],
        out_specs=pl.BlockSpec((2, EB, 128), lambda i: (0, i, 0)),
        out_shape=jax.ShapeDtypeStruct((2, E, 128), jnp.float32),
    )(f, w1s, b1s, w2, b2)
    return hfs


def _sc_body(xt2, hf2, src_h, dst_h, out, srcv, dstv, rows, hfv, agg, sem):
    c = lax.axis_index("c")
    s = lax.axis_index("s")

    def zrow(i, carry):
        for j in range(8):
            rows[i, pl.ds(j * 16, 16)] = jnp.zeros((16,), jnp.float32)
        return carry

    lax.fori_loop(0, KE, zrow, 0)

    def zchunk(t, carry):
        pltpu.sync_copy(rows, agg.at[pl.ds(s * ZR + t * KE, KE)])
        return carry

    lax.fori_loop(0, ZR // KE, zchunk, 0)
    plsc.subcore_barrier()

    base = s * EPT
    coff = c * N

    def chunk(t, carry):
        off = base + t * KE
        pltpu.sync_copy(src_h.at[pl.ds(off, KE)], srcv)
        pltpu.sync_copy(dst_h.at[pl.ds(off, KE)], dstv)
        for j in range(KE // 16):
            sl = pl.ds(j * 16, 16)
            srcv[sl] = srcv[sl] + coff
        pltpu.async_copy(xt2.at[srcv], rows, sem).wait()
        pltpu.sync_copy(hf2.at[pl.ds(c * E + off, KE)], hfv)

        def mrow(i, icarry):
            for j in range(8):
                sl = pl.ds(j * 16, 16)
                rows[i, sl] = rows[i, sl] * hfv[i, sl]
            return icarry

        lax.fori_loop(0, KE, mrow, 0)
        pltpu.sync_copy(rows, agg.at[dstv], add=True)
        return carry

    lax.fori_loop(0, NCH, chunk, 0)
    plsc.subcore_barrier()
    pltpu.sync_copy(agg.at[pl.ds(s * ZR, ZR)], out.at[c, pl.ds(s * ZR, ZR)])


def _sc_conv(xt2, hf2, src, dst):
    mesh = plsc.VectorSubcoreMesh(core_axis_name="c", subcore_axis_name="s")
    k = functools.partial(
        pl.kernel,
        out_type=jax.ShapeDtypeStruct((2, NP, 128), jnp.float32),
        mesh=mesh,
        scratch_types=[
            pltpu.VMEM((KE,), jnp.int32),
            pltpu.VMEM((KE,), jnp.int32),
            pltpu.VMEM((KE, 128), jnp.float32),
            pltpu.VMEM((KE, 128), jnp.float32),
            pltpu.VMEM_SHARED((NP, 128), jnp.float32),
            pltpu.SemaphoreType.DMA,
        ],
    )(_sc_body)
    return k(xt2, hf2, src, dst)


def _c1_body(ag1_ref, ag2_ref, xt_ref, bc_ref, relw_ref, relb_ref, rootw_ref,
             catw_ref, catb_ref, l0w_ref, l0b_ref, l1w_ref, l1b_ref,
             l2w_ref, l2b_ref, h_ref, seg_ref, cnt_ref):
    i = pl.program_id(0)
    xt = xt_ref[...]
    root = _mm(xt, rootw_ref[...])
    a1 = (_mm(ag1_ref[0], relw_ref[0:128, :]) +
          _mm(ag1_ref[1], relw_ref[128:256, :]))
    h1 = jnp.maximum(a1 + relb_ref[...] + root, 0.0)
    a2 = (_mm(ag2_ref[0], relw_ref[0:128, :]) +
          _mm(ag2_ref[1], relw_ref[128:256, :]))
    h2 = jnp.maximum(a2 + relb_ref[...] + root, 0.0)
    h = jnp.maximum(_mm(h1, catw_ref[0:256, :]) + _mm(h2, catw_ref[256:512, :])
                    + catb_ref[...], 0.0)
    h = h + xt
    for wr, br in ((l0w_ref, l0b_ref), (l1w_ref, l1b_ref), (l2w_ref, l2b_ref)):
        h = jnp.maximum(_mm(h, wr[...]) + br[...], 0.0) + h
    h_ref[...] = h

    oh = (bc_ref[...] ==
          lax.broadcasted_iota(jnp.int32, (RB, G), 1).astype(jnp.float32)
          ).astype(jnp.float32)

    @pl.when(i == 0)
    def _():
        seg_ref[...] = jnp.zeros_like(seg_ref)
        cnt_ref[...] = jnp.zeros_like(cnt_ref)

    seg_ref[...] += _mmT(oh, h)
    cnt_ref[0:1, :] += jnp.sum(oh, axis=0, keepdims=True)


def _c2_body(h_ref, bc_ref, seg_ref, cnt_ref, ms_ref, sub_ref, vseg_ref):
    i = pl.program_id(0)
    oh = (bc_ref[...] ==
          lax.broadcasted_iota(jnp.int32, (RB, G), 1).astype(jnp.float32)
          ).astype(jnp.float32)
    recip = 1.0 / jnp.maximum(cnt_ref[0:1, :], 1.0)
    meanb = _mm(oh * recip, seg_ref[...])
    sub = h_ref[...] - meanb * ms_ref[...]
    sub_ref[...] = sub

    @pl.when(i == 0)
    def _():
        vseg_ref[...] = jnp.zeros_like(vseg_ref)

    vseg_ref[...] += _mmT(oh, sub * sub)


def _c3_body(sub_ref, bc_ref, vseg_ref, cnt_ref, gw_ref, gb_ref,
             finw_ref, finb_ref, out_ref):
    oh = (bc_ref[...] ==
          lax.broadcasted_iota(jnp.int32, (RB, G), 1).astype(jnp.float32)
          ).astype(jnp.float32)
    recip = 1.0 / jnp.maximum(cnt_ref[0:1, :], 1.0)
    varb = _mm(oh * recip, vseg_ref[...])
    sub = sub_ref[...]
    hh = gw_ref[...] * sub * lax.rsqrt(varb + EPS) + gb_ref[...]
    out_ref[...] = _mm(hh, finw_ref[...]) + finb_ref[...]


def _stage_c(ag1, ag2, xt, bcol, relw, relb, rootw, catw, catb,
             lws, gn_w, gn_b, gn_ms, finw, finb):
    l0w, l0b, l1w, l1b, l2w, l2b = lws
    nb = N // RB
    row = lambda i: (i, 0)
    const2 = lambda i: (0, 0)
    wspec = lambda r: pl.BlockSpec((r, H), const2)
    vspec = pl.BlockSpec((1, H), const2)
    h, seg, cnt = pl.pallas_call(
        _c1_body,
        grid=(nb,),
        in_specs=[pl.BlockSpec((2, RB, 128), lambda i: (0, i, 0)),
                  pl.BlockSpec((2, RB, 128), lambda i: (0, i, 0)),
                  pl.BlockSpec((RB, H), row),
                  pl.BlockSpec((RB, 1), row),
                  wspec(H), vspec, wspec(H),
                  wspec(2 * H), vspec,
                  wspec(H), vspec, wspec(H), vspec, wspec(H), vspec],
        out_specs=[pl.BlockSpec((RB, H), row),
                   pl.BlockSpec((G, H), const2),
                   pl.BlockSpec((8, G), const2)],
        out_shape=[jax.ShapeDtypeStruct((N, H), jnp.float32),
                   jax.ShapeDtypeStruct((G, H), jnp.float32),
                   jax.ShapeDtypeStruct((8, G), jnp.float32)],
    )(ag1, ag2, xt, bcol, relw, relb, rootw, catw, catb,
      l0w, l0b, l1w, l1b, l2w, l2b)
    sub, vseg = pl.pallas_call(
        _c2_body,
        grid=(nb,),
        in_specs=[pl.BlockSpec((RB, H), row),
                  pl.BlockSpec((RB, 1), row),
                  pl.BlockSpec((G, H), const2),
                  pl.BlockSpec((8, G), const2),
                  vspec],
        out_specs=[pl.BlockSpec((RB, H), row),
                   pl.BlockSpec((G, H), const2)],
        out_shape=[jax.ShapeDtypeStruct((N, H), jnp.float32),
                   jax.ShapeDtypeStruct((G, H), jnp.float32)],
    )(h, bcol, seg, cnt, gn_ms)
    out = pl.pallas_call(
        _c3_body,
        grid=(nb,),
        in_specs=[pl.BlockSpec((RB, H), row),
                  pl.BlockSpec((RB, 1), row),
                  pl.BlockSpec((G, H), const2),
                  pl.BlockSpec((8, G), const2),
                  vspec, vspec, wspec(H), vspec],
        out_specs=pl.BlockSpec((RB, H), row),
        out_shape=jax.ShapeDtypeStruct((N, H), jnp.float32),
    )(sub, bcol, vseg, cnt, gn_w, gn_b, finw, finb)
    return out


def kernel(x, edge_index, edge_attr, feature1, feature2, batch,
           lin_x_W, lin_x_b, lin_x_g, lin_x_be,
           f1_W1, f1_b1, f1_g, f1_be, f1_W2, f1_b2,
           f2_W1, f2_b1, f2_g, f2_be, f2_W2, f2_b2,
           c1_rel_W, c1_rel_b, c1_root_W,
           cat_W, cat_b,
           l0_W, l0_b, l1_W, l1_b, l2_W, l2_b,
           gn_w, gn_b, gn_ms,
           fin_W, fin_b):
    del edge_attr, lin_x_b, f1_b1, f2_b1
    row = lambda v: v.reshape(1, H)

    xt, xts = _stage_a(x, lin_x_W, row(lin_x_g), row(lin_x_be))

    f1p = jnp.pad(feature1, ((0, 0), (0, 10)))
    w1p = jnp.pad(f1_W1, ((0, 10), (0, 0)))
    hfs1 = _stage_b(f1p, w1p, row(f1_g), row(f1_be), f1_W2, row(f1_b2))
    f2p = jnp.pad(feature2, ((0, 0), (0, 14)))
    w2p = jnp.pad(f2_W1, ((0, 14), (0, 0)))
    hfs2 = _stage_b(f2p, w2p, row(f2_g), row(f2_be), f2_W2, row(f2_b2))

    xt2 = xts.reshape(2 * N, 128)
    src = edge_index[0]
    dst = edge_index[1]
    ag1 = _sc_conv(xt2, hfs1.reshape(2 * E, 128), src, dst)
    ag2 = _sc_conv(xt2, hfs2.reshape(2 * E, 128), src, dst)

    bcol = batch.astype(jnp.float32).reshape(N, 1)
    return _stage_c(ag1, ag2, xt, bcol,
                    c1_rel_W, row(c1_rel_b), c1_root_W,
                    cat_W, row(cat_b),
                    (l0_W, row(l0_b), l1_W, row(l1_b), l2_W, row(l2_b)),
                    row(gn_w), row(gn_b), row(gn_ms),
                    fin_W, row(fin_b))


# speedup vs baseline: 2.8103x; 1.7028x over previous
import functools

import jax
import jax.numpy as jnp
from jax import lax
from jax.experimental import pallas as pl
from jax.experimental.pallas import tpu as pltpu
from jax.experimental.pallas import tpu_sc as plsc

N = 10000
E = 160000
H = 256
G = 64
NP = 10240
RB = 1000
EB = 1600
GB = 2000
KE = 40
EPT = E // 16
NCH = EPT // KE
GRP = 50
GEPT = GRP * KE
NGRP = NCH // GRP
ZR = 640
EPS = 1e-5


def _mm(a, b):
    return jax.lax.dot_general(a, b, (((1,), (0,)), ((), ())),
                               preferred_element_type=jnp.float32)


def _mmT(a, b):
    return jax.lax.dot_general(a, b, (((0,), (0,)), ((), ())),
                               preferred_element_type=jnp.float32)


def _a1_body(x_ref, w_ref, z_ref, st_ref):
    i = pl.program_id(0)
    z = _mm(x_ref[...], w_ref[...])
    z_ref[...] = z

    @pl.when(i == 0)
    def _():
        st_ref[...] = jnp.zeros_like(st_ref)

    st_ref[0:1, :] += jnp.sum(z, axis=0, keepdims=True)
    st_ref[1:2, :] += jnp.sum(z * z, axis=0, keepdims=True)


def _a2_body(z_ref, st_ref, g_ref, be_ref, xt_ref, xts_ref):
    mu = st_ref[0:1, :] / N
    var = st_ref[1:2, :] / N - mu * mu
    xt = jnp.maximum(g_ref[...] * (z_ref[...] - mu) * lax.rsqrt(var + EPS)
                     + be_ref[...], 0.0)
    xt_ref[...] = xt
    xts_ref[0] = xt[:, :128]
    xts_ref[1] = xt[:, 128:]


def _stage_a(x, w, g, be):
    z, st = pl.pallas_call(
        _a1_body,
        grid=(N // RB,),
        in_specs=[pl.BlockSpec((RB, H), lambda i: (i, 0)),
                  pl.BlockSpec((H, H), lambda i: (0, 0))],
        out_specs=[pl.BlockSpec((RB, H), lambda i: (i, 0)),
                   pl.BlockSpec((8, H), lambda i: (0, 0))],
        out_shape=[jax.ShapeDtypeStruct((N, H), jnp.float32),
                   jax.ShapeDtypeStruct((8, H), jnp.float32)],
    )(x, w)
    xt, xts = pl.pallas_call(
        _a2_body,
        grid=(N // RB,),
        in_specs=[pl.BlockSpec((RB, H), lambda i: (i, 0)),
                  pl.BlockSpec((8, H), lambda i: (0, 0)),
                  pl.BlockSpec((1, H), lambda i: (0, 0)),
                  pl.BlockSpec((1, H), lambda i: (0, 0))],
        out_specs=[pl.BlockSpec((RB, H), lambda i: (i, 0)),
                   pl.BlockSpec((2, RB, 128), lambda i: (0, i, 0))],
        out_shape=[jax.ShapeDtypeStruct((N, H), jnp.float32),
                   jax.ShapeDtypeStruct((2, N, 128), jnp.float32)],
    )(z, st, g, be)
    return xt, xts


def _b0_body(f_ref, cs_ref, gr_ref):
    i = pl.program_id(0)
    blk = f_ref[...]

    @pl.when(i == 0)
    def _():
        cs_ref[...] = jnp.zeros_like(cs_ref)
        gr_ref[...] = jnp.zeros_like(gr_ref)

    cs_ref[0:1, :] += jnp.sum(blk, axis=0, keepdims=True)
    gr_ref[...] += _mmT(blk, blk)


def _b1_body(cs_ref, gr_ref, w1_ref, g_ref, be_ref, w1s_ref, b1s_ref):
    mean_f = cs_ref[0:1, :] / E
    mu1 = _mm(mean_f, w1_ref[...])
    cov = gr_ref[...] / E - _mmT(mean_f, mean_f)
    cw = _mm(cov, w1_ref[...])
    var = jnp.sum(cw * w1_ref[...], axis=0, keepdims=True)
    s = g_ref[...] * lax.rsqrt(var + EPS)
    w1s_ref[...] = w1_ref[...] * s
    b1s_ref[...] = be_ref[...] - mu1 * s


def _b2_body(f_ref, w1s_ref, b1s_ref, w2_ref, b2_ref, hfs_ref):
    h1 = jnp.maximum(_mm(f_ref[...], w1s_ref[...]) + b1s_ref[...], 0.0)
    hf = jnp.maximum(_mm(h1, w2_ref[...]) + b2_ref[...], 0.0)
    hfs_ref[0] = hf[:, :128]
    hfs_ref[1] = hf[:, 128:]


def _stage_b(f, w1, g, be, w2, b2):
    dp = f.shape[1]
    cs, gr = pl.pallas_call(
        _b0_body,
        grid=(E // GB,),
        in_specs=[pl.BlockSpec((GB, dp), lambda i: (i, 0))],
        out_specs=[pl.BlockSpec((8, dp), lambda i: (0, 0)),
                   pl.BlockSpec((dp, dp), lambda i: (0, 0))],
        out_shape=[jax.ShapeDtypeStruct((8, dp), jnp.float32),
                   jax.ShapeDtypeStruct((dp, dp), jnp.float32)],
    )(f)
    w1s, b1s = pl.pallas_call(
        _b1_body,
        in_specs=[pl.BlockSpec((8, dp), lambda: (0, 0)),
                  pl.BlockSpec((dp, dp), lambda: (0, 0)),
                  pl.BlockSpec((dp, H), lambda: (0, 0)),
                  pl.BlockSpec((1, H), lambda: (0, 0)),
                  pl.BlockSpec((1, H), lambda: (0, 0))],
        out_specs=[pl.BlockSpec((dp, H), lambda: (0, 0)),
                   pl.BlockSpec((1, H), lambda: (0, 0))],
        out_shape=[jax.ShapeDtypeStruct((dp, H), jnp.float32),
                   jax.ShapeDtypeStruct((1, H), jnp.float32)],
    )(cs, gr, w1, g, be)
    hfs = pl.pallas_call(
        _b2_body,
        grid=(E // EB,),
        in_specs=[pl.BlockSpec((EB, dp), lambda i: (i, 0)),
                  pl.BlockSpec((dp, H), lambda i: (0, 0)),
                  pl.BlockSpec((1, H), lambda i: (0, 0)),
                  pl.BlockSpec((H, H), lambda i: (0, 0)),
                  pl.BlockSpec((1, H), lambda i: (0, 0))],
        out_specs=pl.BlockSpec((2, EB, 128), lambda i: (0, i, 0)),
        out_shape=jax.ShapeDtypeStruct((2, E, 128), jnp.float32),
    )(f, w1s, b1s, w2, b2)
    return hfs


def _sc_body(xt2, hf2, src_h, dst_h, out, srcv, dstv, rows, hfv, agg,
             sg0, sg1, sh0, sh1):
    c = lax.axis_index("c")
    s = lax.axis_index("s")
    sgs = (sg0, sg1)
    shs = (sh0, sh1)
    coff = c * N

    def zrow(i, carry):
        for j in range(8):
            rows[0, i, pl.ds(j * 16, 16)] = jnp.zeros((16,), jnp.float32)
        return carry

    lax.fori_loop(0, KE, zrow, 0)

    def zchunk(t, carry):
        pltpu.sync_copy(rows.at[0], agg.at[pl.ds(s * ZR + t * KE, KE)])
        return carry

    lax.fori_loop(0, ZR // KE, zchunk, 0)
    plsc.subcore_barrier()

    def issue(g, cl, b):
        pltpu.async_copy(xt2.at[srcv.at[pl.ds(cl * KE, KE)]], rows.at[b],
                         sgs[b])
        pltpu.async_copy(hf2.at[pl.ds(c * E + s * EPT + g * GEPT + cl * KE,
                                      KE)],
                         hfv.at[b], shs[b])

    def drain(cl, b):
        pltpu.make_async_copy(xt2.at[pl.ds(0, KE)], rows.at[b], sgs[b]).wait()
        pltpu.make_async_copy(hf2.at[pl.ds(0, KE)], hfv.at[b], shs[b]).wait()

        @plsc.parallel_loop(0, KE, 1, unroll=4)
        def _(i):
            for j in range(8):
                sl = pl.ds(j * 16, 16)
                rows[b, i, sl] = rows[b, i, sl] * hfv[b, i, sl]

        pltpu.sync_copy(rows.at[b], agg.at[dstv.at[cl]], add=True)

    def group(g, carry):
        pltpu.sync_copy(src_h.at[pl.ds(s * EPT + g * GEPT, GEPT)], srcv)
        pltpu.sync_copy(dst_h.at[s, g], dstv)

        def arow(i, icarry):
            sl = pl.ds(i * 16, 16)
            srcv[sl] = srcv[sl] + coff
            return icarry

        lax.fori_loop(0, GEPT // 16, arow, 0)
        issue(g, 0, 0)

        def pair(p, pcarry):
            cl = 2 * p
            issue(g, cl + 1, 1)
            drain(cl, 0)
            issue(g, cl + 2, 0)
            drain(cl + 1, 1)
            return pcarry

        lax.fori_loop(0, GRP // 2 - 1, pair, 0)
        issue(g, GRP - 1, 1)
        drain(GRP - 2, 0)
        drain(GRP - 1, 1)
        return carry

    lax.fori_loop(0, NGRP, group, 0)
    plsc.subcore_barrier()
    pltpu.sync_copy(agg.at[pl.ds(s * ZR, ZR)], out.at[c, pl.ds(s * ZR, ZR)])


def _sc_conv(xt2, hf2, src3, dst3):
    mesh = plsc.VectorSubcoreMesh(core_axis_name="c", subcore_axis_name="s")
    k = functools.partial(
        pl.kernel,
        out_type=jax.ShapeDtypeStruct((2, NP, 128), jnp.float32),
        mesh=mesh,
        scratch_types=[
            pltpu.VMEM((GEPT,), jnp.int32),
            pltpu.VMEM((GRP, KE), jnp.int32),
            pltpu.VMEM((2, KE, 128), jnp.float32),
            pltpu.VMEM((2, KE, 128), jnp.float32),
            pltpu.VMEM_SHARED((NP, 128), jnp.float32),
            pltpu.SemaphoreType.DMA,
            pltpu.SemaphoreType.DMA,
            pltpu.SemaphoreType.DMA,
            pltpu.SemaphoreType.DMA,
        ],
    )(_sc_body)
    return k(xt2, hf2, src3, dst3)


def _c1_body(ag1_ref, ag2_ref, xt_ref, bc_ref, relw_ref, relb_ref, rootw_ref,
             catw_ref, catb_ref, l0w_ref, l0b_ref, l1w_ref, l1b_ref,
             l2w_ref, l2b_ref, h_ref, seg_ref, cnt_ref):
    i = pl.program_id(0)
    xt = xt_ref[...]
    root = _mm(xt, rootw_ref[...])
    a1 = (_mm(ag1_ref[0], relw_ref[0:128, :]) +
          _mm(ag1_ref[1], relw_ref[128:256, :]))
    h1 = jnp.maximum(a1 + relb_ref[...] + root, 0.0)
    a2 = (_mm(ag2_ref[0], relw_ref[0:128, :]) +
          _mm(ag2_ref[1], relw_ref[128:256, :]))
    h2 = jnp.maximum(a2 + relb_ref[...] + root, 0.0)
    h = jnp.maximum(_mm(h1, catw_ref[0:256, :]) + _mm(h2, catw_ref[256:512, :])
                    + catb_ref[...], 0.0)
    h = h + xt
    for wr, br in ((l0w_ref, l0b_ref), (l1w_ref, l1b_ref), (l2w_ref, l2b_ref)):
        h = jnp.maximum(_mm(h, wr[...]) + br[...], 0.0) + h
    h_ref[...] = h

    oh = (bc_ref[...] ==
          lax.broadcasted_iota(jnp.int32, (RB, G), 1).astype(jnp.float32)
          ).astype(jnp.float32)

    @pl.when(i == 0)
    def _():
        seg_ref[...] = jnp.zeros_like(seg_ref)
        cnt_ref[...] = jnp.zeros_like(cnt_ref)

    seg_ref[...] += _mmT(oh, h)
    cnt_ref[0:1, :] += jnp.sum(oh, axis=0, keepdims=True)


def _c2_body(h_ref, bc_ref, seg_ref, cnt_ref, ms_ref, sub_ref, vseg_ref):
    i = pl.program_id(0)
    oh = (bc_ref[...] ==
          lax.broadcasted_iota(jnp.int32, (RB, G), 1).astype(jnp.float32)
          ).astype(jnp.float32)
    recip = 1.0 / jnp.maximum(cnt_ref[0:1, :], 1.0)
    meanb = _mm(oh * recip, seg_ref[...])
    sub = h_ref[...] - meanb * ms_ref[...]
    sub_ref[...] = sub

    @pl.when(i == 0)
    def _():
        vseg_ref[...] = jnp.zeros_like(vseg_ref)

    vseg_ref[...] += _mmT(oh, sub * sub)


def _c3_body(sub_ref, bc_ref, vseg_ref, cnt_ref, gw_ref, gb_ref,
             finw_ref, finb_ref, out_ref):
    oh = (bc_ref[...] ==
          lax.broadcasted_iota(jnp.int32, (RB, G), 1).astype(jnp.float32)
          ).astype(jnp.float32)
    recip = 1.0 / jnp.maximum(cnt_ref[0:1, :], 1.0)
    varb = _mm(oh * recip, vseg_ref[...])
    sub = sub_ref[...]
    hh = gw_ref[...] * sub * lax.rsqrt(varb + EPS) + gb_ref[...]
    out_ref[...] = _mm(hh, finw_ref[...]) + finb_ref[...]


def _stage_c(ag1, ag2, xt, bcol, relw, relb, rootw, catw, catb,
             lws, gn_w, gn_b, gn_ms, finw, finb):
    l0w, l0b, l1w, l1b, l2w, l2b = lws
    nb = N // RB
    row = lambda i: (i, 0)
    const2 = lambda i: (0, 0)
    wspec = lambda r: pl.BlockSpec((r, H), const2)
    vspec = pl.BlockSpec((1, H), const2)
    h, seg, cnt = pl.pallas_call(
        _c1_body,
        grid=(nb,),
        in_specs=[pl.BlockSpec((2, RB, 128), lambda i: (0, i, 0)),
                  pl.BlockSpec((2, RB, 128), lambda i: (0, i, 0)),
                  pl.BlockSpec((RB, H), row),
                  pl.BlockSpec((RB, 1), row),
                  wspec(H), vspec, wspec(H),
                  wspec(2 * H), vspec,
                  wspec(H), vspec, wspec(H), vspec, wspec(H), vspec],
        out_specs=[pl.BlockSpec((RB, H), row),
                   pl.BlockSpec((G, H), const2),
                   pl.BlockSpec((8, G), const2)],
        out_shape=[jax.ShapeDtypeStruct((N, H), jnp.float32),
                   jax.ShapeDtypeStruct((G, H), jnp.float32),
                   jax.ShapeDtypeStruct((8, G), jnp.float32)],
    )(ag1, ag2, xt, bcol, relw, relb, rootw, catw, catb,
      l0w, l0b, l1w, l1b, l2w, l2b)
    sub, vseg = pl.pallas_call(
        _c2_body,
        grid=(nb,),
        in_specs=[pl.BlockSpec((RB, H), row),
                  pl.BlockSpec((RB, 1), row),
                  pl.BlockSpec((G, H), const2),
                  pl.BlockSpec((8, G), const2),
                  vspec],
        out_specs=[pl.BlockSpec((RB, H), row),
                   pl.BlockSpec((G, H), const2)],
        out_shape=[jax.ShapeDtypeStruct((N, H), jnp.float32),
                   jax.ShapeDtypeStruct((G, H), jnp.float32)],
    )(h, bcol, seg, cnt, gn_ms)
    out = pl.pallas_call(
        _c3_body,
        grid=(nb,),
        in_specs=[pl.BlockSpec((RB, H), row),
                  pl.BlockSpec((RB, 1), row),
                  pl.BlockSpec((G, H), const2),
                  pl.BlockSpec((8, G), const2),
                  vspec, vspec, wspec(H), vspec],
        out_specs=pl.BlockSpec((RB, H), row),
        out_shape=jax.ShapeDtypeStruct((N, H), jnp.float32),
    )(sub, bcol, vseg, cnt, gn_w, gn_b, finw, finb)
    return out


def kernel(x, edge_index, edge_attr, feature1, feature2, batch,
           lin_x_W, lin_x_b, lin_x_g, lin_x_be,
           f1_W1, f1_b1, f1_g, f1_be, f1_W2, f1_b2,
           f2_W1, f2_b1, f2_g, f2_be, f2_W2, f2_b2,
           c1_rel_W, c1_rel_b, c1_root_W,
           cat_W, cat_b,
           l0_W, l0_b, l1_W, l1_b, l2_W, l2_b,
           gn_w, gn_b, gn_ms,
           fin_W, fin_b):
    del edge_attr, lin_x_b, f1_b1, f2_b1
    row = lambda v: v.reshape(1, H)

    xt, xts = _stage_a(x, lin_x_W, row(lin_x_g), row(lin_x_be))

    hfs1 = _stage_b(feature1, f1_W1, row(f1_g), row(f1_be), f1_W2, row(f1_b2))
    hfs2 = _stage_b(feature2, f2_W1, row(f2_g), row(f2_be), f2_W2, row(f2_b2))

    xt2 = xts.reshape(2 * N, 128)
    src3 = edge_index[0]
    dst3 = edge_index[1].reshape(16, NGRP, GRP, KE)
    ag1 = _sc_conv(xt2, hfs1.reshape(2 * E, 128), src3, dst3)
    ag2 = _sc_conv(xt2, hfs2.reshape(2 * E, 128), src3, dst3)

    bcol = batch.astype(jnp.float32).reshape(N, 1)
    return _stage_c(ag1, ag2, xt, bcol,
                    c1_rel_W, row(c1_rel_b), c1_root_W,
                    cat_W, row(cat_b),
                    (l0_W, row(l0_b), l1_W, row(l1_b), l2_W, row(l2_b)),
                    row(gn_w), row(gn_b), row(gn_ms),
                    fin_W, row(fin_b))


# speedup vs baseline: 2.9188x; 1.0386x over previous
import functools

import jax
import jax.numpy as jnp
from jax import lax
from jax.experimental import pallas as pl
from jax.experimental.pallas import tpu as pltpu
from jax.experimental.pallas import tpu_sc as plsc

N = 10000
E = 160000
H = 256
G = 64
NP = 10240
RB = 1000
EB = 1600
GB = 2000
KE = 40
EPT = E // 16
NCH = EPT // KE
GRP = 50
GEPT = GRP * KE
NGRP = NCH // GRP
ZR = 640
EPS = 1e-5


def _mm(a, b):
    return jax.lax.dot_general(a, b, (((1,), (0,)), ((), ())),
                               preferred_element_type=jnp.float32)


def _mmT(a, b):
    return jax.lax.dot_general(a, b, (((0,), (0,)), ((), ())),
                               preferred_element_type=jnp.float32)


def _a1_body(x_ref, w_ref, z_ref, st_ref):
    i = pl.program_id(0)
    z = _mm(x_ref[...], w_ref[...])
    z_ref[...] = z

    @pl.when(i == 0)
    def _():
        st_ref[...] = jnp.zeros_like(st_ref)

    st_ref[0:1, :] += jnp.sum(z, axis=0, keepdims=True)
    st_ref[1:2, :] += jnp.sum(z * z, axis=0, keepdims=True)


def _a2_body(z_ref, st_ref, g_ref, be_ref, xt_ref, xts_ref):
    mu = st_ref[0:1, :] / N
    var = st_ref[1:2, :] / N - mu * mu
    xt = jnp.maximum(g_ref[...] * (z_ref[...] - mu) * lax.rsqrt(var + EPS)
                     + be_ref[...], 0.0)
    xt_ref[...] = xt
    xts_ref[0] = xt[:, :128]
    xts_ref[1] = xt[:, 128:]


def _stage_a(x, w, g, be):
    z, st = pl.pallas_call(
        _a1_body,
        grid=(N // RB,),
        in_specs=[pl.BlockSpec((RB, H), lambda i: (i, 0)),
                  pl.BlockSpec((H, H), lambda i: (0, 0))],
        out_specs=[pl.BlockSpec((RB, H), lambda i: (i, 0)),
                   pl.BlockSpec((8, H), lambda i: (0, 0))],
        out_shape=[jax.ShapeDtypeStruct((N, H), jnp.float32),
                   jax.ShapeDtypeStruct((8, H), jnp.float32)],
    )(x, w)
    xt, xts = pl.pallas_call(
        _a2_body,
        grid=(N // RB,),
        in_specs=[pl.BlockSpec((RB, H), lambda i: (i, 0)),
                  pl.BlockSpec((8, H), lambda i: (0, 0)),
                  pl.BlockSpec((1, H), lambda i: (0, 0)),
                  pl.BlockSpec((1, H), lambda i: (0, 0))],
        out_specs=[pl.BlockSpec((RB, H), lambda i: (i, 0)),
                   pl.BlockSpec((2, RB, 128), lambda i: (0, i, 0))],
        out_shape=[jax.ShapeDtypeStruct((N, H), jnp.float32),
                   jax.ShapeDtypeStruct((2, N, 128), jnp.float32)],
    )(z, st, g, be)
    return xt, xts


def _b0_body(f_ref, cs_ref, gr_ref):
    i = pl.program_id(0)
    blk = f_ref[...]

    @pl.when(i == 0)
    def _():
        cs_ref[...] = jnp.zeros_like(cs_ref)
        gr_ref[...] = jnp.zeros_like(gr_ref)

    cs_ref[0:1, :] += jnp.sum(blk, axis=0, keepdims=True)
    gr_ref[...] += _mmT(blk, blk)


def _b1_body(cs_ref, gr_ref, w1_ref, g_ref, be_ref, w1s_ref, b1s_ref):
    mean_f = cs_ref[0:1, :] / E
    mu1 = _mm(mean_f, w1_ref[...])
    cov = gr_ref[...] / E - _mmT(mean_f, mean_f)
    cw = _mm(cov, w1_ref[...])
    var = jnp.sum(cw * w1_ref[...], axis=0, keepdims=True)
    s = g_ref[...] * lax.rsqrt(var + EPS)
    w1s_ref[...] = w1_ref[...] * s
    b1s_ref[...] = be_ref[...] - mu1 * s


def _b2_body(f_ref, w1s_ref, b1s_ref, w2_ref, b2_ref, hfs_ref):
    h1 = jnp.maximum(_mm(f_ref[...], w1s_ref[...]) + b1s_ref[...], 0.0)
    hf = jnp.maximum(_mm(h1, w2_ref[...]) + b2_ref[...], 0.0)
    hfs_ref[0] = hf[:, :128]
    hfs_ref[1] = hf[:, 128:]


def _stage_b(f, w1, g, be, w2, b2):
    dp = f.shape[1]
    cs, gr = pl.pallas_call(
        _b0_body,
        grid=(E // GB,),
        in_specs=[pl.BlockSpec((GB, dp), lambda i: (i, 0))],
        out_specs=[pl.BlockSpec((8, dp), lambda i: (0, 0)),
                   pl.BlockSpec((dp, dp), lambda i: (0, 0))],
        out_shape=[jax.ShapeDtypeStruct((8, dp), jnp.float32),
                   jax.ShapeDtypeStruct((dp, dp), jnp.float32)],
    )(f)
    w1s, b1s = pl.pallas_call(
        _b1_body,
        in_specs=[pl.BlockSpec((8, dp), lambda: (0, 0)),
                  pl.BlockSpec((dp, dp), lambda: (0, 0)),
                  pl.BlockSpec((dp, H), lambda: (0, 0)),
                  pl.BlockSpec((1, H), lambda: (0, 0)),
                  pl.BlockSpec((1, H), lambda: (0, 0))],
        out_specs=[pl.BlockSpec((dp, H), lambda: (0, 0)),
                   pl.BlockSpec((1, H), lambda: (0, 0))],
        out_shape=[jax.ShapeDtypeStruct((dp, H), jnp.float32),
                   jax.ShapeDtypeStruct((1, H), jnp.float32)],
    )(cs, gr, w1, g, be)
    hfs = pl.pallas_call(
        _b2_body,
        grid=(E // EB,),
        in_specs=[pl.BlockSpec((EB, dp), lambda i: (i, 0)),
                  pl.BlockSpec((dp, H), lambda i: (0, 0)),
                  pl.BlockSpec((1, H), lambda i: (0, 0)),
                  pl.BlockSpec((H, H), lambda i: (0, 0)),
                  pl.BlockSpec((1, H), lambda i: (0, 0))],
        out_specs=pl.BlockSpec((2, EB, 128), lambda i: (0, i, 0)),
        out_shape=jax.ShapeDtypeStruct((2, E, 128), jnp.float32),
    )(f, w1s, b1s, w2, b2)
    return hfs


def _sc_body(xt2, hf2, src_h, dst_h, out, srcv, dstv, rows, hfv, msg, agg,
             sg0, sg1, sh0, sh1, ss0, ss1):
    c = lax.axis_index("c")
    s = lax.axis_index("s")
    sgs = (sg0, sg1)
    shs = (sh0, sh1)
    sss = (ss0, ss1)
    coff = c * N

    def zrow(i, carry):
        for j in range(8):
            rows[0, i, pl.ds(j * 16, 16)] = jnp.zeros((16,), jnp.float32)
        return carry

    lax.fori_loop(0, KE, zrow, 0)

    def zchunk(t, carry):
        pltpu.sync_copy(rows.at[0], agg.at[pl.ds(s * ZR + t * KE, KE)])
        return carry

    lax.fori_loop(0, ZR // KE, zchunk, 0)
    plsc.subcore_barrier()

    def issue(g, cl, b):
        pltpu.async_copy(xt2.at[srcv.at[pl.ds(cl * KE, KE)]], rows.at[b],
                         sgs[b])
        pltpu.async_copy(hf2.at[pl.ds(c * E + s * EPT + g * GEPT + cl * KE,
                                      KE)],
                         hfv.at[b], shs[b])

    def wait_scatter(b):
        pltpu.make_async_copy(msg.at[b], agg.at[pl.ds(0, KE)], sss[b]).wait()

    def drain(cl, b):
        pltpu.make_async_copy(xt2.at[pl.ds(0, KE)], rows.at[b], sgs[b]).wait()
        pltpu.make_async_copy(hf2.at[pl.ds(0, KE)], hfv.at[b], shs[b]).wait()

        @pl.when(jnp.int32(cl) >= 2)
        def _():
            wait_scatter(b)

        @plsc.parallel_loop(0, KE, 1, unroll=4)
        def _(i):
            for j in range(8):
                sl = pl.ds(j * 16, 16)
                msg[b, i, sl] = rows[b, i, sl] * hfv[b, i, sl]

        pltpu.async_copy(msg.at[b], agg.at[dstv.at[cl]], sss[b], add=True)

    def group(g, carry):
        @pl.when(g > 0)
        def _():
            wait_scatter(0)
            wait_scatter(1)

        pltpu.sync_copy(src_h.at[pl.ds(s * EPT + g * GEPT, GEPT)], srcv)
        pltpu.sync_copy(dst_h.at[s, g], dstv)

        def arow(i, icarry):
            sl = pl.ds(i * 16, 16)
            srcv[sl] = srcv[sl] + coff
            return icarry

        lax.fori_loop(0, GEPT // 16, arow, 0)
        issue(g, 0, 0)

        def pair(p, pcarry):
            cl = 2 * p
            issue(g, cl + 1, 1)
            drain(cl, 0)
            issue(g, cl + 2, 0)
            drain(cl + 1, 1)
            return pcarry

        lax.fori_loop(0, GRP // 2 - 1, pair, 0)
        issue(g, GRP - 1, 1)
        drain(GRP - 2, 0)
        drain(GRP - 1, 1)
        return carry

    lax.fori_loop(0, NGRP, group, 0)
    wait_scatter(0)
    wait_scatter(1)
    plsc.subcore_barrier()
    pltpu.sync_copy(agg.at[pl.ds(s * ZR, ZR)], out.at[c, pl.ds(s * ZR, ZR)])


def _sc_conv(xt2, hf2, src3, dst3):
    mesh = plsc.VectorSubcoreMesh(core_axis_name="c", subcore_axis_name="s")
    k = functools.partial(
        pl.kernel,
        out_type=jax.ShapeDtypeStruct((2, NP, 128), jnp.float32),
        mesh=mesh,
        scratch_types=[
            pltpu.VMEM((GEPT,), jnp.int32),
            pltpu.VMEM((GRP, KE), jnp.int32),
            pltpu.VMEM((2, KE, 128), jnp.float32),
            pltpu.VMEM((2, KE, 128), jnp.float32),
            pltpu.VMEM((2, KE, 128), jnp.float32),
            pltpu.VMEM_SHARED((NP, 128), jnp.float32),
            pltpu.SemaphoreType.DMA,
            pltpu.SemaphoreType.DMA,
            pltpu.SemaphoreType.DMA,
            pltpu.SemaphoreType.DMA,
            pltpu.SemaphoreType.DMA,
            pltpu.SemaphoreType.DMA,
        ],
    )(_sc_body)
    return k(xt2, hf2, src3, dst3)


def _c1_body(ag1_ref, ag2_ref, xt_ref, bc_ref, relw_ref, relb_ref, rootw_ref,
             catw_ref, catb_ref, l0w_ref, l0b_ref, l1w_ref, l1b_ref,
             l2w_ref, l2b_ref, h_ref, seg_ref, cnt_ref):
    i = pl.program_id(0)
    xt = xt_ref[...]
    root = _mm(xt, rootw_ref[...])
    a1 = (_mm(ag1_ref[0], relw_ref[0:128, :]) +
          _mm(ag1_ref[1], relw_ref[128:256, :]))
    h1 = jnp.maximum(a1 + relb_ref[...] + root, 0.0)
    a2 = (_mm(ag2_ref[0], relw_ref[0:128, :]) +
          _mm(ag2_ref[1], relw_ref[128:256, :]))
    h2 = jnp.maximum(a2 + relb_ref[...] + root, 0.0)
    h = jnp.maximum(_mm(h1, catw_ref[0:256, :]) + _mm(h2, catw_ref[256:512, :])
                    + catb_ref[...], 0.0)
    h = h + xt
    for wr, br in ((l0w_ref, l0b_ref), (l1w_ref, l1b_ref), (l2w_ref, l2b_ref)):
        h = jnp.maximum(_mm(h, wr[...]) + br[...], 0.0) + h
    h_ref[...] = h

    oh = (bc_ref[...] ==
          lax.broadcasted_iota(jnp.int32, (RB, G), 1).astype(jnp.float32)
          ).astype(jnp.float32)

    @pl.when(i == 0)
    def _():
        seg_ref[...] = jnp.zeros_like(seg_ref)
        cnt_ref[...] = jnp.zeros_like(cnt_ref)

    seg_ref[...] += _mmT(oh, h)
    cnt_ref[0:1, :] += jnp.sum(oh, axis=0, keepdims=True)


def _c2_body(h_ref, bc_ref, seg_ref, cnt_ref, ms_ref, sub_ref, vseg_ref):
    i = pl.program_id(0)
    oh = (bc_ref[...] ==
          lax.broadcasted_iota(jnp.int32, (RB, G), 1).astype(jnp.float32)
          ).astype(jnp.float32)
    recip = 1.0 / jnp.maximum(cnt_ref[0:1, :], 1.0)
    meanb = _mm(oh * recip, seg_ref[...])
    sub = h_ref[...] - meanb * ms_ref[...]
    sub_ref[...] = sub

    @pl.when(i == 0)
    def _():
        vseg_ref[...] = jnp.zeros_like(vseg_ref)

    vseg_ref[...] += _mmT(oh, sub * sub)


def _c3_body(sub_ref, bc_ref, vseg_ref, cnt_ref, gw_ref, gb_ref,
             finw_ref, finb_ref, out_ref):
    oh = (bc_ref[...] ==
          lax.broadcasted_iota(jnp.int32, (RB, G), 1).astype(jnp.float32)
          ).astype(jnp.float32)
    recip = 1.0 / jnp.maximum(cnt_ref[0:1, :], 1.0)
    varb = _mm(oh * recip, vseg_ref[...])
    sub = sub_ref[...]
    hh = gw_ref[...] * sub * lax.rsqrt(varb + EPS) + gb_ref[...]
    out_ref[...] = _mm(hh, finw_ref[...]) + finb_ref[...]


def _stage_c(ag1, ag2, xt, bcol, relw, relb, rootw, catw, catb,
             lws, gn_w, gn_b, gn_ms, finw, finb):
    l0w, l0b, l1w, l1b, l2w, l2b = lws
    nb = N // RB
    row = lambda i: (i, 0)
    const2 = lambda i: (0, 0)
    wspec = lambda r: pl.BlockSpec((r, H), const2)
    vspec = pl.BlockSpec((1, H), const2)
    h, seg, cnt = pl.pallas_call(
        _c1_body,
        grid=(nb,),
        in_specs=[pl.BlockSpec((2, RB, 128), lambda i: (0, i, 0)),
                  pl.BlockSpec((2, RB, 128), lambda i: (0, i, 0)),
                  pl.BlockSpec((RB, H), row),
                  pl.BlockSpec((RB, 1), row),
                  wspec(H), vspec, wspec(H),
                  wspec(2 * H), vspec,
                  wspec(H), vspec, wspec(H), vspec, wspec(H), vspec],
        out_specs=[pl.BlockSpec((RB, H), row),
                   pl.BlockSpec((G, H), const2),
                   pl.BlockSpec((8, G), const2)],
        out_shape=[jax.ShapeDtypeStruct((N, H), jnp.float32),
                   jax.ShapeDtypeStruct((G, H), jnp.float32),
                   jax.ShapeDtypeStruct((8, G), jnp.float32)],
    )(ag1, ag2, xt, bcol, relw, relb, rootw, catw, catb,
      l0w, l0b, l1w, l1b, l2w, l2b)
    sub, vseg = pl.pallas_call(
        _c2_body,
        grid=(nb,),
        in_specs=[pl.BlockSpec((RB, H), row),
                  pl.BlockSpec((RB, 1), row),
                  pl.BlockSpec((G, H), const2),
                  pl.BlockSpec((8, G), const2),
                  vspec],
        out_specs=[pl.BlockSpec((RB, H), row),
                   pl.BlockSpec((G, H), const2)],
        out_shape=[jax.ShapeDtypeStruct((N, H), jnp.float32),
                   jax.ShapeDtypeStruct((G, H), jnp.float32)],
    )(h, bcol, seg, cnt, gn_ms)
    out = pl.pallas_call(
        _c3_body,
        grid=(nb,),
        in_specs=[pl.BlockSpec((RB, H), row),
                  pl.BlockSpec((RB, 1), row),
                  pl.BlockSpec((G, H), const2),
                  pl.BlockSpec((8, G), const2),
                  vspec, vspec, wspec(H), vspec],
        out_specs=pl.BlockSpec((RB, H), row),
        out_shape=jax.ShapeDtypeStruct((N, H), jnp.float32),
    )(sub, bcol, vseg, cnt, gn_w, gn_b, finw, finb)
    return out


def kernel(x, edge_index, edge_attr, feature1, feature2, batch,
           lin_x_W, lin_x_b, lin_x_g, lin_x_be,
           f1_W1, f1_b1, f1_g, f1_be, f1_W2, f1_b2,
           f2_W1, f2_b1, f2_g, f2_be, f2_W2, f2_b2,
           c1_rel_W, c1_rel_b, c1_root_W,
           cat_W, cat_b,
           l0_W, l0_b, l1_W, l1_b, l2_W, l2_b,
           gn_w, gn_b, gn_ms,
           fin_W, fin_b):
    del edge_attr, lin_x_b, f1_b1, f2_b1
    row = lambda v: v.reshape(1, H)

    xt, xts = _stage_a(x, lin_x_W, row(lin_x_g), row(lin_x_be))

    hfs1 = _stage_b(feature1, f1_W1, row(f1_g), row(f1_be), f1_W2, row(f1_b2))
    hfs2 = _stage_b(feature2, f2_W1, row(f2_g), row(f2_be), f2_W2, row(f2_b2))

    xt2 = xts.reshape(2 * N, 128)
    src3 = edge_index[0]
    dst3 = edge_index[1].reshape(16, NGRP, GRP, KE)
    ag1 = _sc_conv(xt2, hfs1.reshape(2 * E, 128), src3, dst3)
    ag2 = _sc_conv(xt2, hfs2.reshape(2 * E, 128), src3, dst3)

    bcol = batch.astype(jnp.float32).reshape(N, 1)
    return _stage_c(ag1, ag2, xt, bcol,
                    c1_rel_W, row(c1_rel_b), c1_root_W,
                    cat_W, row(cat_b),
                    (l0_W, row(l0_b), l1_W, row(l1_b), l2_W, row(l2_b)),
                    row(gn_w), row(gn_b), row(gn_ms),
                    fin_W, row(fin_b))
